# baseline (speedup 1.0000x reference)
import functools

import jax
import jax.numpy as jnp
from jax import lax
from jax.experimental import pallas as pl
from jax.experimental.pallas import tpu as pltpu
from jax.experimental.pallas import tpu_sc as plsc

_NC = 2
_NS = 16
_NW = _NC * _NS
_CHUNK = 128


def _normalize_body(table_ref, out_ref):
    x = table_ref[...]
    norm = jnp.sqrt(jnp.sum(x * x, axis=1, keepdims=True))
    out_ref[...] = x / jnp.maximum(norm, 1e-12)


def _normalize_table(table):
    return pl.pallas_call(
        _normalize_body,
        out_shape=jax.ShapeDtypeStruct(table.shape, table.dtype),
    )(table)


@functools.cache
def _make_sc_gather(B, D, dtype):
    b_per_w = B // _NW
    n_chunks = b_per_w // _CHUNK
    mesh = plsc.VectorSubcoreMesh(
        core_axis_name="c", subcore_axis_name="s",
        num_cores=_NC, num_subcores=_NS)

    @functools.partial(
        pl.kernel,
        out_type=jax.ShapeDtypeStruct((B, D), dtype),
        mesh=mesh,
        scratch_types=[
            pltpu.VMEM((n_chunks, _CHUNK), jnp.int32),
            pltpu.VMEM((b_per_w, D), dtype),
            pltpu.SemaphoreType.DMA((n_chunks,)),
            pltpu.SemaphoreType.DMA,
        ],
    )
    def gather(table_hbm, idx_hbm, out_hbm, idx_v, rows_v, gsems, wsem):
        wid = lax.axis_index("s") * _NC + lax.axis_index("c")
        base = wid * b_per_w
        pltpu.sync_copy(idx_hbm.at[wid], idx_v)
        gathers = [
            pltpu.async_copy(
                table_hbm.at[idx_v.at[j]],
                rows_v.at[pl.ds(j * _CHUNK, _CHUNK)],
                gsems.at[j],
            )
            for j in range(n_chunks)
        ]
        writes = []
        for j in range(n_chunks):
            gathers[j].wait()
            writes.append(pltpu.async_copy(
                rows_v.at[pl.ds(j * _CHUNK, _CHUNK)],
                out_hbm.at[pl.ds(base + j * _CHUNK, _CHUNK)],
                wsem,
            ))
        for w in writes:
            w.wait()

    return gather


def kernel(expert_indices, table):
    B = expert_indices.shape[0]
    D = table.shape[1]
    table_n = _normalize_table(table)
    idx = expert_indices.astype(jnp.int32).reshape(_NW, B // _NW // _CHUNK, _CHUNK)
    return _make_sc_gather(B, D, table.dtype)(table_n, idx)


# speedup vs baseline: 1.5432x; 1.5432x over previous
import functools

import jax
import jax.numpy as jnp
from jax import lax
from jax.experimental import pallas as pl
from jax.experimental.pallas import tpu as pltpu
from jax.experimental.pallas import tpu_sc as plsc

_NC = 2
_NS = 16
_NW = _NC * _NS
_CHUNK = 128


def _normalize_body(table_ref, out_ref):
    x = table_ref[...]
    norm = jnp.sqrt(jnp.sum(x * x, axis=1, keepdims=True))
    out_ref[...] = x / jnp.maximum(norm, 1e-12)


def _normalize_table(table):
    return pl.pallas_call(
        _normalize_body,
        out_shape=jax.ShapeDtypeStruct(table.shape, table.dtype),
    )(table)


@functools.cache
def _make_sc_gather(B, D, dtype):
    b_per_w = B // _NW
    n_chunks = b_per_w // _CHUNK
    mesh = plsc.VectorSubcoreMesh(
        core_axis_name="c", subcore_axis_name="s",
        num_cores=_NC, num_subcores=_NS)

    @functools.partial(
        pl.kernel,
        out_type=jax.ShapeDtypeStruct((B, D), dtype),
        mesh=mesh,
        scratch_types=[
            pltpu.VMEM((n_chunks, _CHUNK), jnp.int32),
            pltpu.VMEM((b_per_w, D), dtype),
            pltpu.VMEM_SHARED((64, D), dtype),
            pltpu.SemaphoreType.DMA((n_chunks,)),
            pltpu.SemaphoreType.DMA,
        ],
    )
    def gather(table_hbm, idx_hbm, out_hbm, idx_v, rows_v, table_sh, gsems, wsem):
        sid = lax.axis_index("s")
        wid = sid * _NC + lax.axis_index("c")
        base = wid * b_per_w
        @pl.when(sid == 0)
        def _():
            pltpu.sync_copy(table_hbm, table_sh)
        pltpu.sync_copy(idx_hbm.at[wid], idx_v)
        plsc.subcore_barrier()
        gathers = [
            pltpu.async_copy(
                table_sh.at[idx_v.at[j]],
                rows_v.at[pl.ds(j * _CHUNK, _CHUNK)],
                gsems.at[j],
            )
            for j in range(n_chunks)
        ]
        writes = []
        for j in range(n_chunks):
            gathers[j].wait()
            writes.append(pltpu.async_copy(
                rows_v.at[pl.ds(j * _CHUNK, _CHUNK)],
                out_hbm.at[pl.ds(base + j * _CHUNK, _CHUNK)],
                wsem,
            ))
        for w in writes:
            w.wait()

    return gather


def kernel(expert_indices, table):
    B = expert_indices.shape[0]
    D = table.shape[1]
    table_n = _normalize_table(table)
    idx = expert_indices.astype(jnp.int32).reshape(_NW, B // _NW // _CHUNK, _CHUNK)
    return _make_sc_gather(B, D, table.dtype)(table_n, idx)


# speedup vs baseline: 1.5841x; 1.0265x over previous
import functools

import jax
import jax.numpy as jnp
from jax import lax
from jax.experimental import pallas as pl
from jax.experimental.pallas import tpu as pltpu
from jax.experimental.pallas import tpu_sc as plsc

_NC = 2
_NS = 16
_NW = _NC * _NS
_CHUNK = 128
_LANES = 16


def _rsqrt16(s):
    i = lax.bitcast_convert_type(s, jnp.int32)
    i = jnp.full((_LANES,), 0x5F3759DF, jnp.int32) - (i >> 1)
    y = lax.bitcast_convert_type(i, jnp.float32)
    for _ in range(3):
        y = y * (1.5 - 0.5 * s * y * y)
    return y


@functools.cache
def _make_sc_kernel(B, E, D, dtype):
    b_per_w = B // _NW
    n_chunks = b_per_w // _CHUNK
    rows_per_sub = E // _NS
    col_chunks = D // _LANES
    mesh = plsc.VectorSubcoreMesh(
        core_axis_name="c", subcore_axis_name="s",
        num_cores=_NC, num_subcores=_NS)

    @functools.partial(
        pl.kernel,
        out_type=jax.ShapeDtypeStruct((B, D), dtype),
        mesh=mesh,
        scratch_types=[
            pltpu.VMEM((n_chunks, _CHUNK), jnp.int32),
            pltpu.VMEM((b_per_w, D), dtype),
            pltpu.VMEM((rows_per_sub, D), dtype),
            pltpu.VMEM_SHARED((E, D), dtype),
            pltpu.SemaphoreType.DMA((n_chunks,)),
            pltpu.SemaphoreType.DMA,
            pltpu.SemaphoreType.DMA,
        ],
    )
    def body(table_hbm, idx_hbm, out_hbm, idx_v, rows_v, traw_v, table_sh,
             gsems, wsem, isem):
        sid = lax.axis_index("s")
        wid = sid * _NC + lax.axis_index("c")
        base = wid * b_per_w

        idx_copy = pltpu.async_copy(idx_hbm.at[wid], idx_v, isem)
        pltpu.sync_copy(table_hbm.at[pl.ds(sid * rows_per_sub, rows_per_sub)],
                        traw_v)
        for r in range(rows_per_sub):
            row = traw_v.at[r]
            chunks = [row[pl.ds(c * _LANES, _LANES)] for c in range(col_chunks)]
            acc = chunks[0] * chunks[0]
            for c in range(1, col_chunks):
                acc = acc + chunks[c] * chunks[c]
            lanes = lax.iota(jnp.int32, _LANES)
            dnums = lax.GatherDimensionNumbers(
                offset_dims=(), collapsed_slice_dims=(0,),
                start_index_map=(0,))
            s16 = acc
            for sh in (8, 4, 2, 1):
                perm = lax.reshape(lanes ^ sh, (_LANES, 1))
                s16 = s16 + lax.gather(
                    s16, perm, dnums, slice_sizes=(1,),
                    mode=lax.GatherScatterMode.PROMISE_IN_BOUNDS)
            inv = jnp.minimum(_rsqrt16(s16), 1e12)
            for c in range(col_chunks):
                row[pl.ds(c * _LANES, _LANES)] = chunks[c] * inv
        pltpu.sync_copy(traw_v,
                        table_sh.at[pl.ds(sid * rows_per_sub, rows_per_sub)])
        plsc.subcore_barrier()
        idx_copy.wait()

        gathers = [
            pltpu.async_copy(
                table_sh.at[idx_v.at[j]],
                rows_v.at[pl.ds(j * _CHUNK, _CHUNK)],
                gsems.at[j],
            )
            for j in range(n_chunks)
        ]
        writes = []
        for j in range(n_chunks):
            gathers[j].wait()
            writes.append(pltpu.async_copy(
                rows_v.at[pl.ds(j * _CHUNK, _CHUNK)],
                out_hbm.at[pl.ds(base + j * _CHUNK, _CHUNK)],
                wsem,
            ))
        for w in writes:
            w.wait()

    return body


def kernel(expert_indices, table):
    B = expert_indices.shape[0]
    E, D = table.shape
    idx = expert_indices.astype(jnp.int32).reshape(_NW, B // _NW // _CHUNK, _CHUNK)
    return _make_sc_kernel(B, E, D, table.dtype)(table, idx)


# speedup vs baseline: 1.6014x; 1.0109x over previous
import functools

import jax
import jax.numpy as jnp
from jax import lax
from jax.experimental import pallas as pl
from jax.experimental.pallas import tpu as pltpu
from jax.experimental.pallas import tpu_sc as plsc

_NC = 2
_NS = 16
_NW = _NC * _NS
_CHUNK = 64
_LANES = 16


def _rsqrt16(s):
    i = lax.bitcast_convert_type(s, jnp.int32)
    i = jnp.full((_LANES,), 0x5F3759DF, jnp.int32) - (i >> 1)
    y = lax.bitcast_convert_type(i, jnp.float32)
    for _ in range(3):
        y = y * (1.5 - 0.5 * s * y * y)
    return y


@functools.cache
def _make_sc_kernel(B, E, D, dtype):
    b_per_w = B // _NW
    n_chunks = b_per_w // _CHUNK
    rows_per_sub = E // _NS
    col_chunks = D // _LANES
    mesh = plsc.VectorSubcoreMesh(
        core_axis_name="c", subcore_axis_name="s",
        num_cores=_NC, num_subcores=_NS)

    @functools.partial(
        pl.kernel,
        out_type=jax.ShapeDtypeStruct((B, D), dtype),
        mesh=mesh,
        scratch_types=[
            pltpu.VMEM((n_chunks, _CHUNK), jnp.int32),
            pltpu.VMEM((b_per_w, D), dtype),
            pltpu.VMEM((rows_per_sub, D), dtype),
            pltpu.VMEM_SHARED((E, D), dtype),
            pltpu.SemaphoreType.DMA((n_chunks,)),
            pltpu.SemaphoreType.DMA,
            pltpu.SemaphoreType.DMA,
        ],
    )
    def body(table_hbm, idx_hbm, out_hbm, idx_v, rows_v, traw_v, table_sh,
             gsems, wsem, isem):
        sid = lax.axis_index("s")
        wid = sid * _NC + lax.axis_index("c")
        base = wid * b_per_w

        idx_copy = pltpu.async_copy(idx_hbm.at[wid], idx_v, isem)
        pltpu.sync_copy(table_hbm.at[pl.ds(sid * rows_per_sub, rows_per_sub)],
                        traw_v)
        for r in range(rows_per_sub):
            row = traw_v.at[r]
            chunks = [row[pl.ds(c * _LANES, _LANES)] for c in range(col_chunks)]
            acc = chunks[0] * chunks[0]
            for c in range(1, col_chunks):
                acc = acc + chunks[c] * chunks[c]
            lanes = lax.iota(jnp.int32, _LANES)
            dnums = lax.GatherDimensionNumbers(
                offset_dims=(), collapsed_slice_dims=(0,),
                start_index_map=(0,))
            s16 = acc
            for sh in (8, 4, 2, 1):
                perm = lax.reshape(lanes ^ sh, (_LANES, 1))
                s16 = s16 + lax.gather(
                    s16, perm, dnums, slice_sizes=(1,),
                    mode=lax.GatherScatterMode.PROMISE_IN_BOUNDS)
            inv = jnp.minimum(_rsqrt16(s16), 1e12)
            for c in range(col_chunks):
                row[pl.ds(c * _LANES, _LANES)] = chunks[c] * inv
        pltpu.sync_copy(traw_v,
                        table_sh.at[pl.ds(sid * rows_per_sub, rows_per_sub)])
        plsc.subcore_barrier()
        idx_copy.wait()

        gathers = [
            pltpu.async_copy(
                table_sh.at[idx_v.at[j]],
                rows_v.at[pl.ds(j * _CHUNK, _CHUNK)],
                gsems.at[j],
            )
            for j in range(n_chunks)
        ]
        writes = []
        for j in range(n_chunks):
            gathers[j].wait()
            writes.append(pltpu.async_copy(
                rows_v.at[pl.ds(j * _CHUNK, _CHUNK)],
                out_hbm.at[pl.ds(base + j * _CHUNK, _CHUNK)],
                wsem,
            ))
        for w in writes:
            w.wait()

    return body


def kernel(expert_indices, table):
    B = expert_indices.shape[0]
    E, D = table.shape
    idx = expert_indices.astype(jnp.int32).reshape(_NW, B // _NW // _CHUNK, _CHUNK)
    return _make_sc_kernel(B, E, D, table.dtype)(table, idx)


# speedup vs baseline: 1.6064x; 1.0031x over previous
import functools

import jax
import jax.numpy as jnp
from jax import lax
from jax.experimental import pallas as pl
from jax.experimental.pallas import tpu as pltpu
from jax.experimental.pallas import tpu_sc as plsc

_NC = 2
_NS = 16
_NW = _NC * _NS
_CHUNK = 32
_LANES = 16


def _rsqrt16(s):
    i = lax.bitcast_convert_type(s, jnp.int32)
    i = jnp.full((_LANES,), 0x5F3759DF, jnp.int32) - (i >> 1)
    y = lax.bitcast_convert_type(i, jnp.float32)
    for _ in range(3):
        y = y * (1.5 - 0.5 * s * y * y)
    return y


@functools.cache
def _make_sc_kernel(B, E, D, dtype):
    b_per_w = B // _NW
    n_chunks = b_per_w // _CHUNK
    rows_per_sub = E // _NS
    col_chunks = D // _LANES
    mesh = plsc.VectorSubcoreMesh(
        core_axis_name="c", subcore_axis_name="s",
        num_cores=_NC, num_subcores=_NS)

    @functools.partial(
        pl.kernel,
        out_type=jax.ShapeDtypeStruct((B, D), dtype),
        mesh=mesh,
        scratch_types=[
            pltpu.VMEM((n_chunks, _CHUNK), jnp.int32),
            pltpu.VMEM((b_per_w, D), dtype),
            pltpu.VMEM((rows_per_sub, D), dtype),
            pltpu.VMEM_SHARED((E, D), dtype),
            pltpu.SemaphoreType.DMA((n_chunks,)),
            pltpu.SemaphoreType.DMA,
            pltpu.SemaphoreType.DMA,
        ],
    )
    def body(table_hbm, idx_hbm, out_hbm, idx_v, rows_v, traw_v, table_sh,
             gsems, wsem, isem):
        sid = lax.axis_index("s")
        wid = sid * _NC + lax.axis_index("c")
        base = wid * b_per_w

        idx_copy = pltpu.async_copy(idx_hbm.at[wid], idx_v, isem)
        pltpu.sync_copy(table_hbm.at[pl.ds(sid * rows_per_sub, rows_per_sub)],
                        traw_v)
        for r in range(rows_per_sub):
            row = traw_v.at[r]
            chunks = [row[pl.ds(c * _LANES, _LANES)] for c in range(col_chunks)]
            acc = chunks[0] * chunks[0]
            for c in range(1, col_chunks):
                acc = acc + chunks[c] * chunks[c]
            lanes = lax.iota(jnp.int32, _LANES)
            dnums = lax.GatherDimensionNumbers(
                offset_dims=(), collapsed_slice_dims=(0,),
                start_index_map=(0,))
            s16 = acc
            for sh in (8, 4, 2, 1):
                perm = lax.reshape(lanes ^ sh, (_LANES, 1))
                s16 = s16 + lax.gather(
                    s16, perm, dnums, slice_sizes=(1,),
                    mode=lax.GatherScatterMode.PROMISE_IN_BOUNDS)
            inv = jnp.minimum(_rsqrt16(s16), 1e12)
            for c in range(col_chunks):
                row[pl.ds(c * _LANES, _LANES)] = chunks[c] * inv
        pltpu.sync_copy(traw_v,
                        table_sh.at[pl.ds(sid * rows_per_sub, rows_per_sub)])
        plsc.subcore_barrier()
        idx_copy.wait()

        gathers = [
            pltpu.async_copy(
                table_sh.at[idx_v.at[j]],
                rows_v.at[pl.ds(j * _CHUNK, _CHUNK)],
                gsems.at[j],
            )
            for j in range(n_chunks)
        ]
        writes = []
        for j in range(n_chunks):
            gathers[j].wait()
            writes.append(pltpu.async_copy(
                rows_v.at[pl.ds(j * _CHUNK, _CHUNK)],
                out_hbm.at[pl.ds(base + j * _CHUNK, _CHUNK)],
                wsem,
            ))
        for w in writes:
            w.wait()

    return body


def kernel(expert_indices, table):
    B = expert_indices.shape[0]
    E, D = table.shape
    idx = expert_indices.astype(jnp.int32).reshape(_NW, B // _NW // _CHUNK, _CHUNK)
    return _make_sc_kernel(B, E, D, table.dtype)(table, idx)
